# baseline (speedup 1.0000x reference)
import functools

import jax
import jax.numpy as jnp
from jax.experimental import pallas as pl
from jax.experimental.pallas import tpu as pltpu

_N = 4096
_B = 16


def _pad128(d):
    return max(128, (d + 127) // 128 * 128)


def _linear_body(x_ref, w_ref, b_ref, o_ref):
    o_ref[...] = (
        jnp.dot(x_ref[...], w_ref[...], preferred_element_type=jnp.float32)
        + b_ref[...]
    )


def _linear(x2, w, bias):
    n = x2.shape[0]
    din, dout = w.shape
    return pl.pallas_call(
        _linear_body,
        grid=(_B,),
        in_specs=[
            pl.BlockSpec((n, din), lambda b: (0, b)),
            pl.BlockSpec((din, dout), lambda b: (0, 0)),
            pl.BlockSpec((1, dout), lambda b: (0, 0)),
        ],
        out_specs=pl.BlockSpec((n, dout), lambda b: (0, b)),
        out_shape=jax.ShapeDtypeStruct((n, _B * dout), jnp.float32),
    )(x2, w, bias.reshape(1, dout))


def _spmm_body(a_ref, z_ref, o_ref, *, k_steps):
    k = pl.program_id(2)

    @pl.when(k == 0)
    def _init():
        o_ref[...] = jnp.zeros_like(o_ref)

    o_ref[...] += jnp.dot(
        a_ref[...].astype(jnp.bfloat16),
        z_ref[...].astype(jnp.bfloat16),
        preferred_element_type=jnp.float32,
    )

    @pl.when(k == k_steps - 1)
    def _relu():
        o_ref[...] = jnp.maximum(o_ref[...], 0.0)


def _spmm_dense(a, z2):
    n = a.shape[0]
    c = z2.shape[1]
    rb = 1024
    kb = 512
    cb = min(c, 2048)
    assert c % cb == 0 and n % rb == 0 and n % kb == 0
    grid = (n // rb, c // cb, n // kb)
    return pl.pallas_call(
        functools.partial(_spmm_body, k_steps=grid[2]),
        grid=grid,
        in_specs=[
            pl.BlockSpec((rb, kb), lambda i, j, k: (i, k)),
            pl.BlockSpec((kb, cb), lambda i, j, k: (k, j)),
        ],
        out_specs=pl.BlockSpec((rb, cb), lambda i, j, k: (i, j)),
        out_shape=jax.ShapeDtypeStruct((n, c), jnp.float32),
    )(a, z2)


def _densify(idx, val):
    rows = idx[:, 0].astype(jnp.int32)
    cols = idx[:, 1].astype(jnp.int32)
    return jnp.zeros((_N, _N), jnp.float32).at[rows, cols].add(val)


def kernel(H, DADsm_indices, DADsm_values, DADsp_indices, DADsp_values,
           W0, b0, W1, b1, W2, b2, W3, b3, W4, b4, W5, b5):
    a_sm = _densify(DADsm_indices, DADsm_values)
    a_sp = _densify(DADsp_indices, DADsp_values)
    ws = [W0, W1, W2, W3, W4, W5]
    bs = [b0, b1, b2, b3, b4, b5]

    wps, bps = [], []
    for w, b in zip(ws, bs):
        dinp, doutp = _pad128(w.shape[0]), _pad128(w.shape[1])
        wps.append(jnp.zeros((dinp, doutp), jnp.float32).at[: w.shape[0], : w.shape[1]].set(w))
        bps.append(jnp.zeros((doutp,), jnp.float32).at[: b.shape[0]].set(b))

    x2 = jnp.transpose(H, (1, 0, 2)).reshape(_N, _B * H.shape[2])
    for layer in range(6):
        a = a_sm if layer < 3 else a_sp
        z2 = _linear(x2, wps[layer], bps[layer])
        x2 = _spmm_dense(a, z2)
    doutp = wps[5].shape[1]
    dout = ws[5].shape[1]
    return jnp.transpose(x2.reshape(_N, _B, doutp), (1, 0, 2))[:, :, :dout]


# speedup vs baseline: 1.1467x; 1.1467x over previous
import functools

import jax
import jax.numpy as jnp
from jax.experimental import pallas as pl
from jax.experimental.pallas import tpu as pltpu

_N = 4096
_B = 16


def _pad128(d):
    return max(128, (d + 127) // 128 * 128)


def _linear_body(x_ref, w_ref, b_ref, o_ref):
    acc = jnp.dot(x_ref[...], w_ref[...], preferred_element_type=jnp.float32)
    o_ref[...] = (acc + b_ref[...]).astype(jnp.bfloat16)


def _linear(x2, w, bias):
    n = x2.shape[0]
    din, dout = w.shape
    return pl.pallas_call(
        _linear_body,
        grid=(_B,),
        in_specs=[
            pl.BlockSpec((n, din), lambda b: (0, b)),
            pl.BlockSpec((din, dout), lambda b: (0, 0)),
            pl.BlockSpec((1, dout), lambda b: (0, 0)),
        ],
        out_specs=pl.BlockSpec((n, dout), lambda b: (0, b)),
        out_shape=jax.ShapeDtypeStruct((n, _B * dout), jnp.bfloat16),
    )(x2, w, bias.reshape(1, dout))


def _spmm_body(a_ref, z_ref, o_ref, acc_ref, *, k_steps):
    k = pl.program_id(2)

    @pl.when(k == 0)
    def _init():
        acc_ref[...] = jnp.zeros_like(acc_ref)

    acc_ref[...] += jnp.dot(
        a_ref[...], z_ref[...], preferred_element_type=jnp.float32
    )

    @pl.when(k == k_steps - 1)
    def _relu():
        o_ref[...] = jnp.maximum(acc_ref[...], 0.0).astype(jnp.bfloat16)


def _spmm_dense(a, z2):
    n = a.shape[0]
    c = z2.shape[1]
    rb = 2048
    kb = 512
    cb = min(c, 2048)
    assert c % cb == 0 and n % rb == 0 and n % kb == 0
    grid = (n // rb, c // cb, n // kb)
    return pl.pallas_call(
        functools.partial(_spmm_body, k_steps=grid[2]),
        grid=grid,
        in_specs=[
            pl.BlockSpec((rb, kb), lambda i, j, k: (i, k)),
            pl.BlockSpec((kb, cb), lambda i, j, k: (k, j)),
        ],
        out_specs=pl.BlockSpec((rb, cb), lambda i, j, k: (i, j)),
        out_shape=jax.ShapeDtypeStruct((n, c), jnp.bfloat16),
        scratch_shapes=[pltpu.VMEM((rb, cb), jnp.float32)],
    )(a, z2)


def _densify(idx, val):
    rows = idx[:, 0].astype(jnp.int32)
    cols = idx[:, 1].astype(jnp.int32)
    dense = jnp.zeros((_N, _N), jnp.float32).at[rows, cols].add(val)
    return dense.astype(jnp.bfloat16)


def kernel(H, DADsm_indices, DADsm_values, DADsp_indices, DADsp_values,
           W0, b0, W1, b1, W2, b2, W3, b3, W4, b4, W5, b5):
    a_sm = _densify(DADsm_indices, DADsm_values)
    a_sp = _densify(DADsp_indices, DADsp_values)
    ws = [W0, W1, W2, W3, W4, W5]
    bs = [b0, b1, b2, b3, b4, b5]

    wps, bps = [], []
    for w, b in zip(ws, bs):
        dinp, doutp = _pad128(w.shape[0]), _pad128(w.shape[1])
        wps.append(
            jnp.zeros((dinp, doutp), jnp.float32)
            .at[: w.shape[0], : w.shape[1]].set(w).astype(jnp.bfloat16)
        )
        bps.append(jnp.zeros((doutp,), jnp.float32).at[: b.shape[0]].set(b))

    x2 = jnp.transpose(H, (1, 0, 2)).reshape(_N, _B * H.shape[2])
    x2 = x2.astype(jnp.bfloat16)
    for layer in range(6):
        a = a_sm if layer < 3 else a_sp
        z2 = _linear(x2, wps[layer], bps[layer])
        x2 = _spmm_dense(a, z2)
    doutp = wps[5].shape[1]
    dout = ws[5].shape[1]
    out = x2.astype(jnp.float32)
    return jnp.transpose(out.reshape(_N, _B, doutp), (1, 0, 2))[:, :, :dout]


# speedup vs baseline: 1.1512x; 1.0039x over previous
import functools

import jax
import jax.numpy as jnp
from jax.experimental import pallas as pl
from jax.experimental.pallas import tpu as pltpu

_N = 4096
_B = 16


def _pad128(d):
    return max(128, (d + 127) // 128 * 128)


def _linear_body(x_ref, w_ref, b_ref, o_ref):
    acc = jnp.dot(x_ref[...], w_ref[...], preferred_element_type=jnp.float32)
    o_ref[...] = (acc + b_ref[...]).astype(jnp.bfloat16)


def _linear0_body(h_ref, w_ref, b_ref, o_ref):
    x = h_ref[0].astype(jnp.bfloat16)
    acc = jnp.dot(x, w_ref[...], preferred_element_type=jnp.float32)
    o_ref[...] = (acc + b_ref[...]).astype(jnp.bfloat16)


def _linear0(h, w, bias):
    _, n, f = h.shape
    din, dout = w.shape
    assert f == din
    return pl.pallas_call(
        _linear0_body,
        grid=(_B,),
        in_specs=[
            pl.BlockSpec((1, n, din), lambda b: (b, 0, 0)),
            pl.BlockSpec((din, dout), lambda b: (0, 0)),
            pl.BlockSpec((1, dout), lambda b: (0, 0)),
        ],
        out_specs=pl.BlockSpec((n, dout), lambda b: (0, b)),
        out_shape=jax.ShapeDtypeStruct((n, _B * dout), jnp.bfloat16),
    )(h, w, bias.reshape(1, dout))


def _linear(x2, w, bias):
    n = x2.shape[0]
    din, dout = w.shape
    return pl.pallas_call(
        _linear_body,
        grid=(_B,),
        in_specs=[
            pl.BlockSpec((n, din), lambda b: (0, b)),
            pl.BlockSpec((din, dout), lambda b: (0, 0)),
            pl.BlockSpec((1, dout), lambda b: (0, 0)),
        ],
        out_specs=pl.BlockSpec((n, dout), lambda b: (0, b)),
        out_shape=jax.ShapeDtypeStruct((n, _B * dout), jnp.bfloat16),
    )(x2, w, bias.reshape(1, dout))


def _spmm_body(a_ref, z_ref, o_ref, acc_ref, *, k_steps):
    k = pl.program_id(2)

    @pl.when(k == 0)
    def _init():
        acc_ref[...] = jnp.zeros_like(acc_ref)

    acc_ref[...] += jnp.dot(
        a_ref[...], z_ref[...], preferred_element_type=jnp.float32
    )

    @pl.when(k == k_steps - 1)
    def _relu():
        o_ref[...] = jnp.maximum(acc_ref[...], 0.0).astype(jnp.bfloat16)


def _spmm_dense(a, z2):
    n = a.shape[0]
    c = z2.shape[1]
    rb = 2048
    kb = 512
    cb = min(c, 2048)
    assert c % cb == 0 and n % rb == 0 and n % kb == 0
    grid = (n // rb, c // cb, n // kb)
    return pl.pallas_call(
        functools.partial(_spmm_body, k_steps=grid[2]),
        grid=grid,
        in_specs=[
            pl.BlockSpec((rb, kb), lambda i, j, k: (i, k)),
            pl.BlockSpec((kb, cb), lambda i, j, k: (k, j)),
        ],
        out_specs=pl.BlockSpec((rb, cb), lambda i, j, k: (i, j)),
        out_shape=jax.ShapeDtypeStruct((n, c), jnp.bfloat16),
        scratch_shapes=[pltpu.VMEM((rb, cb), jnp.float32)],
    )(a, z2)


def _densify(idx, val):
    rows = idx[:, 0].astype(jnp.int32)
    cols = idx[:, 1].astype(jnp.int32)
    dense = jnp.zeros((_N, _N), jnp.float32).at[rows, cols].add(val)
    return dense.astype(jnp.bfloat16)


def kernel(H, DADsm_indices, DADsm_values, DADsp_indices, DADsp_values,
           W0, b0, W1, b1, W2, b2, W3, b3, W4, b4, W5, b5):
    a_sm = _densify(DADsm_indices, DADsm_values)
    a_sp = _densify(DADsp_indices, DADsp_values)
    ws = [W0, W1, W2, W3, W4, W5]
    bs = [b0, b1, b2, b3, b4, b5]

    wps, bps = [], []
    for w, b in zip(ws, bs):
        dinp, doutp = _pad128(w.shape[0]), _pad128(w.shape[1])
        wps.append(
            jnp.zeros((dinp, doutp), jnp.float32)
            .at[: w.shape[0], : w.shape[1]].set(w).astype(jnp.bfloat16)
        )
        bps.append(jnp.zeros((doutp,), jnp.float32).at[: b.shape[0]].set(b))

    x2 = _spmm_dense(a_sm, _linear0(H, wps[0], bps[0]))
    for layer in range(1, 6):
        a = a_sm if layer < 3 else a_sp
        z2 = _linear(x2, wps[layer], bps[layer])
        x2 = _spmm_dense(a, z2)
    doutp = wps[5].shape[1]
    dout = ws[5].shape[1]
    out = x2.astype(jnp.float32)
    return jnp.transpose(out.reshape(_N, _B, doutp), (1, 0, 2))[:, :, :dout]
